# baseline (speedup 1.0000x reference)
import functools

import jax
import jax.numpy as jnp
from jax import lax
from jax.experimental import pallas as pl
from jax.experimental.pallas import tpu as pltpu
from jax.experimental.pallas import tpu_sc as plsc

_B, _P, _O = 16, 16384, 8
_R, _C = 128, 128
_THRESHOLD = 0.35
_NEG_POS = 7
_VAR0, _VAR1 = 0.1, 0.2
_LANES = 16
_CHUNKS = _P // _LANES


def _smooth_l1(x, y):
    d = jnp.abs(x - y)
    return jnp.where(d < 1.0, 0.5 * d * d, d - 0.5)


def _match_kernel(conf_ref, priors_ref, tgt_ref,
                  hnm_out, k_out, bsel_out, part_out, acc):
    b = pl.program_id(0)

    @pl.when(b == 0)
    def _init():
        acc[0] = 0.0
        acc[1] = 0.0

    pcx = priors_ref[0]
    pcy = priors_ref[1]
    pw = priors_ref[2]
    ph = priors_ref[3]
    px1 = pcx - pw * 0.5
    py1 = pcy - ph * 0.5
    px2 = pcx + pw * 0.5
    py2 = pcy + ph * 0.5
    area_p = pw * ph
    pidx = (lax.broadcasted_iota(jnp.int32, (_R, _C), 0) * _C
            + lax.broadcasted_iota(jnp.int32, (_R, _C), 1))

    bto = jnp.full((_R, _C), -1.0, jnp.float32)
    bti = jnp.zeros((_R, _C), jnp.int32)
    best_prior = []
    for o in range(_O):
        tx1 = tgt_ref[0, o, 0]
        ty1 = tgt_ref[0, o, 1]
        tx2 = tgt_ref[0, o, 2]
        ty2 = tgt_ref[0, o, 3]
        iw = jnp.maximum(jnp.minimum(px2, tx2) - jnp.maximum(px1, tx1), 0.0)
        ih = jnp.maximum(jnp.minimum(py2, ty2) - jnp.maximum(py1, ty1), 0.0)
        inter = iw * ih
        area_t = (tx2 - tx1) * (ty2 - ty1)
        iou = inter / (area_t + area_p - inter + 1e-12)
        m = jnp.max(iou)
        best_prior.append(jnp.min(jnp.where(iou == m, pidx, _P)))
        upd = iou > bto
        bto = jnp.where(upd, iou, bto)
        bti = jnp.where(upd, o, bti)

    for o in range(_O):
        forced = pidx == best_prior[o]
        bto = jnp.where(forced, 2.0, bto)
        bti = jnp.where(forced, o, bti)

    pos = bto >= _THRESHOLD
    posf = pos.astype(jnp.float32)
    npos_row = jnp.sum(posf)
    bsel_out[0] = bti + jnp.where(pos, _O, 0)

    c0 = conf_ref[0, 0]
    c1 = conf_ref[0, 1]
    mx = jnp.maximum(c0, c1)
    lse = mx + jnp.log(jnp.exp(c0 - mx) + jnp.exp(c1 - mx))
    ce = lse - jnp.where(pos, c1, c0)
    pos_ce_row = jnp.sum(ce * posf)
    hnm_out[0] = jnp.where(pos, 0.0, ce)
    kval = jnp.minimum(_NEG_POS * npos_row, float(_P - 1))
    k_out[0, 0, :] = jnp.full((_LANES,), kval, jnp.float32)

    acc[0] = acc[0] + pos_ce_row
    acc[1] = acc[1] + npos_row

    @pl.when(b == _B - 1)
    def _fin():
        part_out[0, 0] = acc[0]
        part_out[0, 1] = acc[1]


def _loc_landm_kernel(loc_ref, landm_ref, priors_ref, tgt_ref, bsel_ref,
                      part_out, acc):
    b = pl.program_id(0)

    @pl.when(b == 0)
    def _init():
        acc[0] = 0.0
        acc[1] = 0.0

    pcx = priors_ref[0]
    pcy = priors_ref[1]
    pw = priors_ref[2]
    ph = priors_ref[3]
    px1 = pcx - pw * 0.5
    py1 = pcy - ph * 0.5
    px2 = pcx + pw * 0.5
    py2 = pcy + ph * 0.5

    bsel = bsel_ref[0]
    pos = bsel >= _O
    posf = pos.astype(jnp.float32)
    bti = bsel - jnp.where(pos, _O, 0)

    zero = jnp.zeros((_R, _C), jnp.float32)
    mt = [zero] * 4
    ml = [zero] * 8
    for o in range(_O):
        sel = bti == o
        for k in range(4):
            mt[k] = jnp.where(sel, tgt_ref[0, o, k], mt[k])
        for k in range(8):
            ml[k] = jnp.where(sel, tgt_ref[0, o, 4 + k], ml[k])

    g0 = ((mt[0] + mt[2]) * 0.5 - pcx) / (_VAR0 * pw)
    g1 = ((mt[1] + mt[3]) * 0.5 - pcy) / (_VAR0 * ph)
    g2 = jnp.log(jnp.maximum((mt[2] - mt[0]) / pw, 1e-8)) / _VAR1
    g3 = jnp.log(jnp.maximum((mt[3] - mt[1]) / ph, 1e-8)) / _VAR1
    loss_l_row = (jnp.sum(_smooth_l1(loc_ref[0, 0], g0) * posf)
                  + jnp.sum(_smooth_l1(loc_ref[0, 1], g1) * posf)
                  + jnp.sum(_smooth_l1(loc_ref[0, 2], g2) * posf)
                  + jnp.sum(_smooth_l1(loc_ref[0, 3], g3) * posf))

    ax0 = jnp.maximum(landm_ref[0, 0], 0.0)
    ax1 = landm_ref[0, 1]
    ax2 = landm_ref[0, 2]
    ay0 = landm_ref[0, 3]
    ay1 = jnp.maximum(landm_ref[0, 4], 0.0)
    ay2 = landm_ref[0, 5]
    loss_m_row = 0.0
    corners = ((px1, py1), (px2, py1), (px1, py2), (px2, py2))
    for k, (cx, cy) in enumerate(corners):
        u = cx * 2.0 - 1.0
        v = cy * 2.0 - 1.0
        outx = (ax0 * u + ax1 * v + ax2 + 1.0) * 0.5
        outy = (ay0 * u + ay1 * v + ay2 + 1.0) * 0.5
        loss_m_row += jnp.sum(_smooth_l1(outx, ml[2 * k]) * posf)
        loss_m_row += jnp.sum(_smooth_l1(outy, ml[2 * k + 1]) * posf)

    acc[0] = acc[0] + loss_l_row
    acc[1] = acc[1] + loss_m_row

    @pl.when(b == _B - 1)
    def _fin():
        part_out[0, 0] = acc[0]
        part_out[0, 1] = acc[1]


def _lane_red(v, op):
    idx = lax.iota(jnp.int32, _LANES)
    for s in (1, 2, 4, 8):
        v = op(v, v[jnp.bitwise_xor(idx, s)])
    return v


def _sc_topk_kernel(hnmf_hbm, hnmi_hbm, k_hbm, out_hbm,
                    rowf_v, rowi_v, kv_v, out_v):
    cid = lax.axis_index("c")
    sid = lax.axis_index("s")
    wid = sid * 2 + cid
    row = lax.rem(wid, _B)

    pltpu.sync_copy(hnmf_hbm.at[row], rowf_v)
    pltpu.sync_copy(hnmi_hbm.at[row], rowi_v)
    pltpu.sync_copy(k_hbm.at[row, 0], kv_v)
    kfv = kv_v[...]
    kiv = kfv.astype(jnp.int32)

    def bit_step(i, tv):
        shiftv = jnp.full((_LANES,), 30 - i, jnp.int32)
        candv = tv + lax.shift_left(jnp.full((_LANES,), 1, jnp.int32),
                                    shiftv)

        def cnt_step(j, a):
            bits = rowi_v[pl.ds(j * _LANES, _LANES)]
            return a + jnp.where(bits >= candv, 1, 0)

        a = lax.fori_loop(0, _CHUNKS, cnt_step,
                          jnp.zeros((_LANES,), jnp.int32), unroll=8)
        cntv = _lane_red(a, jnp.add)
        return jnp.where(cntv >= kiv, candv, tv)

    tv = lax.fori_loop(0, 31, bit_step, jnp.zeros((_LANES,), jnp.int32))

    def fin_step(j, carry):
        sgt, cgt, eqm = carry
        v = rowf_v[pl.ds(j * _LANES, _LANES)]
        bits = rowi_v[pl.ds(j * _LANES, _LANES)]
        gt = bits > tv
        sgt = sgt + jnp.where(gt, v, 0.0)
        cgt = cgt + jnp.where(gt, 1, 0)
        eqm = jnp.maximum(eqm, jnp.where(bits == tv, v, -1.0))
        return sgt, cgt, eqm

    sgt, cgt, eqm = lax.fori_loop(
        0, _CHUNKS, fin_step,
        (jnp.zeros((_LANES,), jnp.float32),
         jnp.zeros((_LANES,), jnp.int32),
         jnp.full((_LANES,), -1.0, jnp.float32)), unroll=8)
    tfv = _lane_red(eqm, jnp.maximum)
    topkv = (_lane_red(sgt, jnp.add)
             + (kfv - _lane_red(cgt, jnp.add).astype(jnp.float32)) * tfv)
    out_v[...] = topkv
    pltpu.sync_copy(out_v, out_hbm.at[wid])


def _combine_kernel(part1_ref, part2_ref, topk_ref, out_l, out_c, out_m):
    topk_total = jnp.sum(topk_ref[:, 0])
    n = jnp.maximum(part1_ref[0, 1], 1.0)
    out_l[0, 0] = part2_ref[0, 0] / n
    out_c[0, 0] = (part1_ref[0, 0] + topk_total) / n
    out_m[0, 0] = part2_ref[0, 1] / n


def kernel(loc_data, conf_data, landm_data, priors, targets, epoch, images):
    del epoch, images
    locT = loc_data.transpose(0, 2, 1).reshape(_B, 4, _R, _C)
    confT = conf_data.transpose(0, 2, 1).reshape(_B, 2, _R, _C)
    landmT = landm_data.transpose(0, 2, 1).reshape(_B, 6, _R, _C)
    priorsT = priors.T.reshape(4, _R, _C)

    hnm, kplane, bsel, part1 = pl.pallas_call(
        _match_kernel,
        grid=(_B,),
        in_specs=[
            pl.BlockSpec((1, 2, _R, _C), lambda b: (b, 0, 0, 0)),
            pl.BlockSpec((4, _R, _C), lambda b: (0, 0, 0)),
            pl.BlockSpec((1, _O, 21), lambda b: (b, 0, 0),
                         memory_space=pltpu.SMEM),
        ],
        out_specs=[
            pl.BlockSpec((1, _R, _C), lambda b: (b, 0, 0)),
            pl.BlockSpec((1, 1, _LANES), lambda b: (b, 0, 0)),
            pl.BlockSpec((1, _R, _C), lambda b: (b, 0, 0)),
            pl.BlockSpec((1, 8), lambda b: (0, 0),
                         memory_space=pltpu.SMEM),
        ],
        out_shape=[
            jax.ShapeDtypeStruct((_B, _R, _C), jnp.float32),
            jax.ShapeDtypeStruct((_B, 1, _LANES), jnp.float32),
            jax.ShapeDtypeStruct((_B, _R, _C), jnp.int32),
            jax.ShapeDtypeStruct((1, 8), jnp.float32),
        ],
        scratch_shapes=[pltpu.SMEM((8,), jnp.float32)],
    )(confT, priorsT, targets)

    hnm2 = hnm.reshape(_B, _P)
    hnm2_i = lax.bitcast_convert_type(hnm2, jnp.int32)
    mesh = plsc.VectorSubcoreMesh(core_axis_name="c", subcore_axis_name="s")
    topk = functools.partial(
        pl.kernel,
        mesh=mesh,
        out_type=jax.ShapeDtypeStruct((2 * _B, _LANES), jnp.float32),
        scratch_types=[
            pltpu.VMEM((_P,), jnp.float32),
            pltpu.VMEM((_P,), jnp.int32),
            pltpu.VMEM((_LANES,), jnp.float32),
            pltpu.VMEM((_LANES,), jnp.float32),
        ],
    )(_sc_topk_kernel)(hnm2, hnm2_i, kplane)[:_B]

    part2 = pl.pallas_call(
        _loc_landm_kernel,
        grid=(_B,),
        in_specs=[
            pl.BlockSpec((1, 4, _R, _C), lambda b: (b, 0, 0, 0)),
            pl.BlockSpec((1, 6, _R, _C), lambda b: (b, 0, 0, 0)),
            pl.BlockSpec((4, _R, _C), lambda b: (0, 0, 0)),
            pl.BlockSpec((1, _O, 21), lambda b: (b, 0, 0),
                         memory_space=pltpu.SMEM),
            pl.BlockSpec((1, _R, _C), lambda b: (b, 0, 0)),
        ],
        out_specs=[pl.BlockSpec((1, 8), lambda b: (0, 0),
                                memory_space=pltpu.SMEM)],
        out_shape=[jax.ShapeDtypeStruct((1, 8), jnp.float32)],
        scratch_shapes=[pltpu.SMEM((8,), jnp.float32)],
    )(locT, landmT, priorsT, targets, bsel)[0]

    topk_total = jnp.sum(topk[:, 0])
    n = jnp.maximum(part1[0, 1], 1.0)
    return (part2[0, 0] / n, (part1[0, 0] + topk_total) / n,
            part2[0, 1] / n)


# speedup vs baseline: 1.0716x; 1.0716x over previous
import functools

import jax
import jax.numpy as jnp
from jax import lax
from jax.experimental import pallas as pl
from jax.experimental.pallas import tpu as pltpu
from jax.experimental.pallas import tpu_sc as plsc

_B, _P, _O = 16, 16384, 8
_R, _C = 128, 128
_THRESHOLD = 0.35
_NEG_POS = 7
_VAR0, _VAR1 = 0.1, 0.2
_LANES = 16
_CHUNKS = _P // _LANES


def _smooth_l1(x, y):
    d = jnp.abs(x - y)
    return jnp.where(d < 1.0, 0.5 * d * d, d - 0.5)


def _match_kernel(conf_ref, priors_ref, tgt_ref,
                  hnm_out, k_out, bsel_out, part_out, acc):
    b = pl.program_id(0)

    @pl.when(b == 0)
    def _init():
        acc[0] = 0.0
        acc[1] = 0.0

    pcx = priors_ref[0]
    pcy = priors_ref[1]
    pw = priors_ref[2]
    ph = priors_ref[3]
    px1 = pcx - pw * 0.5
    py1 = pcy - ph * 0.5
    px2 = pcx + pw * 0.5
    py2 = pcy + ph * 0.5
    area_p = pw * ph
    pidx = (lax.broadcasted_iota(jnp.int32, (_R, _C), 0) * _C
            + lax.broadcasted_iota(jnp.int32, (_R, _C), 1))

    bto = jnp.full((_R, _C), -1.0, jnp.float32)
    bti = jnp.zeros((_R, _C), jnp.int32)
    best_prior = []
    for o in range(_O):
        tx1 = tgt_ref[0, o, 0]
        ty1 = tgt_ref[0, o, 1]
        tx2 = tgt_ref[0, o, 2]
        ty2 = tgt_ref[0, o, 3]
        iw = jnp.maximum(jnp.minimum(px2, tx2) - jnp.maximum(px1, tx1), 0.0)
        ih = jnp.maximum(jnp.minimum(py2, ty2) - jnp.maximum(py1, ty1), 0.0)
        inter = iw * ih
        area_t = (tx2 - tx1) * (ty2 - ty1)
        iou = inter / (area_t + area_p - inter + 1e-12)
        m = jnp.max(iou)
        best_prior.append(jnp.min(jnp.where(iou == m, pidx, _P)))
        upd = iou > bto
        bto = jnp.where(upd, iou, bto)
        bti = jnp.where(upd, o, bti)

    for o in range(_O):
        forced = pidx == best_prior[o]
        bto = jnp.where(forced, 2.0, bto)
        bti = jnp.where(forced, o, bti)

    pos = bto >= _THRESHOLD
    posf = pos.astype(jnp.float32)
    npos_row = jnp.sum(posf)
    bsel_out[0] = bti + jnp.where(pos, _O, 0)

    c0 = conf_ref[0, 0]
    c1 = conf_ref[0, 1]
    mx = jnp.maximum(c0, c1)
    lse = mx + jnp.log(jnp.exp(c0 - mx) + jnp.exp(c1 - mx))
    ce = lse - jnp.where(pos, c1, c0)
    pos_ce_row = jnp.sum(ce * posf)
    hnm_out[0] = jnp.where(pos, 0.0, ce)
    kval = jnp.minimum(_NEG_POS * npos_row, float(_P - 1))
    k_out[0, 0, :] = jnp.full((_LANES,), kval, jnp.float32)

    acc[0] = acc[0] + pos_ce_row
    acc[1] = acc[1] + npos_row

    @pl.when(b == _B - 1)
    def _fin():
        part_out[0, 0] = acc[0]
        part_out[0, 1] = acc[1]


def _loc_landm_kernel(loc_ref, landm_ref, priors_ref, tgt_ref, bsel_ref,
                      part_out, acc):
    b = pl.program_id(0)

    @pl.when(b == 0)
    def _init():
        acc[0] = 0.0
        acc[1] = 0.0

    pcx = priors_ref[0]
    pcy = priors_ref[1]
    pw = priors_ref[2]
    ph = priors_ref[3]
    px1 = pcx - pw * 0.5
    py1 = pcy - ph * 0.5
    px2 = pcx + pw * 0.5
    py2 = pcy + ph * 0.5

    bsel = bsel_ref[0]
    pos = bsel >= _O
    posf = pos.astype(jnp.float32)
    bti = bsel - jnp.where(pos, _O, 0)

    zero = jnp.zeros((_R, _C), jnp.float32)
    mt = [zero] * 4
    ml = [zero] * 8
    for o in range(_O):
        sel = bti == o
        for k in range(4):
            mt[k] = jnp.where(sel, tgt_ref[0, o, k], mt[k])
        for k in range(8):
            ml[k] = jnp.where(sel, tgt_ref[0, o, 4 + k], ml[k])

    g0 = ((mt[0] + mt[2]) * 0.5 - pcx) / (_VAR0 * pw)
    g1 = ((mt[1] + mt[3]) * 0.5 - pcy) / (_VAR0 * ph)
    g2 = jnp.log(jnp.maximum((mt[2] - mt[0]) / pw, 1e-8)) / _VAR1
    g3 = jnp.log(jnp.maximum((mt[3] - mt[1]) / ph, 1e-8)) / _VAR1
    loss_l_row = (jnp.sum(_smooth_l1(loc_ref[0, 0], g0) * posf)
                  + jnp.sum(_smooth_l1(loc_ref[0, 1], g1) * posf)
                  + jnp.sum(_smooth_l1(loc_ref[0, 2], g2) * posf)
                  + jnp.sum(_smooth_l1(loc_ref[0, 3], g3) * posf))

    ax0 = jnp.maximum(landm_ref[0, 0], 0.0)
    ax1 = landm_ref[0, 1]
    ax2 = landm_ref[0, 2]
    ay0 = landm_ref[0, 3]
    ay1 = jnp.maximum(landm_ref[0, 4], 0.0)
    ay2 = landm_ref[0, 5]
    loss_m_row = 0.0
    corners = ((px1, py1), (px2, py1), (px1, py2), (px2, py2))
    for k, (cx, cy) in enumerate(corners):
        u = cx * 2.0 - 1.0
        v = cy * 2.0 - 1.0
        outx = (ax0 * u + ax1 * v + ax2 + 1.0) * 0.5
        outy = (ay0 * u + ay1 * v + ay2 + 1.0) * 0.5
        loss_m_row += jnp.sum(_smooth_l1(outx, ml[2 * k]) * posf)
        loss_m_row += jnp.sum(_smooth_l1(outy, ml[2 * k + 1]) * posf)

    acc[0] = acc[0] + loss_l_row
    acc[1] = acc[1] + loss_m_row

    @pl.when(b == _B - 1)
    def _fin():
        part_out[0, 0] = acc[0]
        part_out[0, 1] = acc[1]


def _lane_red(v, op):
    idx = lax.iota(jnp.int32, _LANES)
    for s in (1, 2, 4, 8):
        v = op(v, v[jnp.bitwise_xor(idx, s)])
    return v


def _sc_topk_kernel(hnmf_hbm, hnmi_hbm, k_hbm, out_hbm,
                    rowf_v, rowi_v, kv_v, out_v):
    cid = lax.axis_index("c")
    sid = lax.axis_index("s")
    wid = sid * 2 + cid
    row = lax.rem(wid, _B)

    pltpu.sync_copy(hnmf_hbm.at[row], rowf_v)
    pltpu.sync_copy(hnmi_hbm.at[row], rowi_v)
    pltpu.sync_copy(k_hbm.at[row, 0], kv_v)
    kfv = kv_v[...]
    kiv = kfv.astype(jnp.int32)

    def bit_step(i, tv):
        shiftv = jnp.full((_LANES,), 30 - i, jnp.int32)
        candv = tv + lax.shift_left(jnp.full((_LANES,), 1, jnp.int32),
                                    shiftv)

        def cnt_step(j, a):
            bits = rowi_v[pl.ds(j * _LANES, _LANES)]
            return a + jnp.where(bits >= candv, 1, 0)

        a = lax.fori_loop(0, _CHUNKS, cnt_step,
                          jnp.zeros((_LANES,), jnp.int32), unroll=8)
        cntv = _lane_red(a, jnp.add)
        return jnp.where(cntv >= kiv, candv, tv)

    tv = lax.fori_loop(0, 31, bit_step, jnp.zeros((_LANES,), jnp.int32))

    def fin_step(j, carry):
        sgt, cgt, eqm = carry
        v = rowf_v[pl.ds(j * _LANES, _LANES)]
        bits = rowi_v[pl.ds(j * _LANES, _LANES)]
        gt = bits > tv
        sgt = sgt + jnp.where(gt, v, 0.0)
        cgt = cgt + jnp.where(gt, 1, 0)
        eqm = jnp.maximum(eqm, jnp.where(bits == tv, v, -1.0))
        return sgt, cgt, eqm

    sgt, cgt, eqm = lax.fori_loop(
        0, _CHUNKS, fin_step,
        (jnp.zeros((_LANES,), jnp.float32),
         jnp.zeros((_LANES,), jnp.int32),
         jnp.full((_LANES,), -1.0, jnp.float32)), unroll=8)
    tfv = _lane_red(eqm, jnp.maximum)
    topkv = (_lane_red(sgt, jnp.add)
             + (kfv - _lane_red(cgt, jnp.add).astype(jnp.float32)) * tfv)
    out_v[...] = topkv
    pltpu.sync_copy(out_v, out_hbm.at[wid])


def _combine_kernel(part1_ref, part2_ref, topk_ref, out_l, out_c, out_m):
    topk_total = jnp.sum(topk_ref[:, 0])
    n = jnp.maximum(part1_ref[0, 1], 1.0)
    out_l[0, 0] = part2_ref[0, 0] / n
    out_c[0, 0] = (part1_ref[0, 0] + topk_total) / n
    out_m[0, 0] = part2_ref[0, 1] / n


def kernel(loc_data, conf_data, landm_data, priors, targets, epoch, images):
    del epoch, images
    locT = loc_data.transpose(0, 2, 1).reshape(_B, 4, _R, _C)
    confT = conf_data.transpose(0, 2, 1).reshape(_B, 2, _R, _C)
    landmT = landm_data.transpose(0, 2, 1).reshape(_B, 6, _R, _C)
    priorsT = priors.T.reshape(4, _R, _C)

    hnm, kplane, bsel, part1 = pl.pallas_call(
        _match_kernel,
        grid=(_B,),
        in_specs=[
            pl.BlockSpec((1, 2, _R, _C), lambda b: (b, 0, 0, 0)),
            pl.BlockSpec((4, _R, _C), lambda b: (0, 0, 0)),
            pl.BlockSpec((1, _O, 21), lambda b: (b, 0, 0),
                         memory_space=pltpu.SMEM),
        ],
        out_specs=[
            pl.BlockSpec((1, _R, _C), lambda b: (b, 0, 0)),
            pl.BlockSpec((1, 1, _LANES), lambda b: (b, 0, 0)),
            pl.BlockSpec((1, _R, _C), lambda b: (b, 0, 0)),
            pl.BlockSpec((1, 8), lambda b: (0, 0),
                         memory_space=pltpu.SMEM),
        ],
        out_shape=[
            jax.ShapeDtypeStruct((_B, _R, _C), jnp.float32),
            jax.ShapeDtypeStruct((_B, 1, _LANES), jnp.float32),
            jax.ShapeDtypeStruct((_B, _R, _C), jnp.int32),
            jax.ShapeDtypeStruct((1, 8), jnp.float32),
        ],
        scratch_shapes=[pltpu.SMEM((8,), jnp.float32)],
    )(confT, priorsT, targets)

    hnm2 = hnm.reshape(_B, _P)
    hnm2_i = lax.bitcast_convert_type(hnm2, jnp.int32)
    mesh = plsc.VectorSubcoreMesh(core_axis_name="c", subcore_axis_name="s")
    topk = functools.partial(
        pl.kernel,
        mesh=mesh,
        out_type=jax.ShapeDtypeStruct((2 * _B, _LANES), jnp.float32),
        scratch_types=[
            pltpu.VMEM((_P,), jnp.float32),
            pltpu.VMEM((_P,), jnp.int32),
            pltpu.VMEM((_LANES,), jnp.float32),
            pltpu.VMEM((_LANES,), jnp.float32),
        ],
    )(_sc_topk_kernel)(hnm2, hnm2_i, kplane)[:_B]

    part2 = pl.pallas_call(
        _loc_landm_kernel,
        grid=(_B,),
        in_specs=[
            pl.BlockSpec((1, 4, _R, _C), lambda b: (b, 0, 0, 0)),
            pl.BlockSpec((1, 6, _R, _C), lambda b: (b, 0, 0, 0)),
            pl.BlockSpec((4, _R, _C), lambda b: (0, 0, 0)),
            pl.BlockSpec((1, _O, 21), lambda b: (b, 0, 0),
                         memory_space=pltpu.SMEM),
            pl.BlockSpec((1, _R, _C), lambda b: (b, 0, 0)),
        ],
        out_specs=[pl.BlockSpec((1, 8), lambda b: (0, 0),
                                memory_space=pltpu.SMEM)],
        out_shape=[jax.ShapeDtypeStruct((1, 8), jnp.float32)],
        scratch_shapes=[pltpu.SMEM((8,), jnp.float32)],
    )(locT, landmT, priorsT, targets, bsel)[0]

    smem_out = pl.BlockSpec(memory_space=pltpu.SMEM)
    out_l, out_c, out_m = pl.pallas_call(
        _combine_kernel,
        in_specs=[
            pl.BlockSpec(memory_space=pltpu.SMEM),
            pl.BlockSpec(memory_space=pltpu.SMEM),
            pl.BlockSpec((_B, _LANES), lambda: (0, 0)),
        ],
        out_specs=[smem_out, smem_out, smem_out],
        out_shape=[jax.ShapeDtypeStruct((1, 1), jnp.float32)] * 3,
    )(part1, part2, topk)
    return (out_l.reshape(()), out_c.reshape(()), out_m.reshape(()))
